# Initial kernel scaffold; baseline (speedup 1.0000x reference)
import jax
import jax.numpy as jnp
from jax.experimental import pallas as pl


def kernel(x, edge_index, rows_emb, cols_emb, init_emb, pre_W0, pre_b0, pre_Wr, pre_br, msg_W0, msg_b0, msg_Wr, msg_br, post_W0, post_b0, post_Wr, post_br, lstm_Wih, lstm_Whh, lstm_bih, lstm_bhh, out_W, out_b):
    raise NotImplementedError("write your pallas kernel here")


# speedup vs baseline: 55.9695x; 55.9695x over previous
import functools

import jax
import jax.numpy as jnp
from jax.experimental import pallas as pl
from jax.experimental.pallas import tpu as pltpu

EMBED = 32
LINEAR = 32
LSTM = 32
MSG = 32
N_STEPS = 4
P = 81
DEG = 20
E = P * DEG
BB = 8


def _rrn_block(*refs):
    (xt_ref, gs_ref, r_ref, embk_ref, cellb_ref,
     prek_ref, preb_ref,
     w0sk_ref, w0dk_ref, msgb0_ref, msgk_ref, msgb_ref,
     postak_ref, postxk_ref, postb0_ref, postk_ref, postb_ref,
     wihk_ref, whhk_ref, lstmb_ref,
     outk_ref, outb_ref, out_ref) = refs
    S = BB * LINEAR
    f32 = jnp.float32
    dot = functools.partial(jnp.dot, preferred_element_type=f32)

    xb = xt_ref[0]
    xrep = dot(xb, r_ref[...])
    vals = (jax.lax.broadcasted_iota(jnp.int32, (P, BB * 10), 1) % 10).astype(f32)
    oh = (xrep == vals).astype(f32)
    z = jax.nn.relu(dot(oh, embk_ref[...]) + cellb_ref[...])
    preb = preb_ref[...]
    z = jax.nn.relu(dot(z, prek_ref[0]) + preb[0][None])
    z = jax.nn.relu(dot(z, prek_ref[1]) + preb[1][None])
    x0 = dot(z, prek_ref[2]) + preb[2][None]

    cur = x0
    hs = jnp.zeros((P, S), f32)
    cs = jnp.zeros((P, S), f32)

    gs = gs_ref[...]
    msgb = msgb_ref[...]
    postb = postb_ref[...]
    lstmb = lstmb_ref[...]

    for _ in range(N_STEPS):
        a = dot(cur, w0sk_ref[...])
        b = dot(cur, w0dk_ref[...]) + msgb0_ref[...]
        ae = dot(gs, a)
        be = jnp.broadcast_to(b[:, None, :], (P, DEG, S)).reshape(E, S)
        m = jax.nn.relu(ae + be)
        m = jax.nn.relu(dot(m, msgk_ref[0]) + msgb[0][None])
        m = jax.nn.relu(dot(m, msgk_ref[1]) + msgb[1][None])
        m = dot(m, msgk_ref[2]) + msgb[2][None]
        agg = m.reshape(P, DEG, S).sum(axis=1)

        z = jax.nn.relu(dot(agg, postak_ref[...]) + dot(x0, postxk_ref[...])
                        + postb0_ref[...])
        z = jax.nn.relu(dot(z, postk_ref[0]) + postb[0][None])
        z = jax.nn.relu(dot(z, postk_ref[1]) + postb[1][None])
        z = dot(z, postk_ref[2]) + postb[2][None]

        gates = dot(z, wihk_ref[...]) + dot(hs, whhk_ref[...]) + lstmb
        ig = jax.nn.sigmoid(gates[:, 0 * S:1 * S])
        fg = jax.nn.sigmoid(gates[:, 1 * S:2 * S])
        gg = jnp.tanh(gates[:, 2 * S:3 * S])
        og = jax.nn.sigmoid(gates[:, 3 * S:4 * S])
        cs = fg * cs + ig * gg
        hs = og * jnp.tanh(cs)
        cur = cs

    out_ref[0] = dot(cur, outk_ref[...]) + outb_ref[...]


def kernel(x, edge_index, rows_emb, cols_emb, init_emb, pre_W0, pre_b0,
           pre_Wr, pre_br, msg_W0, msg_b0, msg_Wr, msg_br, post_W0, post_b0,
           post_Wr, post_br, lstm_Wih, lstm_Whh, lstm_bih, lstm_bhh,
           out_W, out_b):
    f32 = jnp.float32
    batch = x.shape[0]
    G = batch // BB
    S = BB * LINEAR
    eye = jnp.eye(BB, dtype=f32)

    def kron(w):
        k, f = w.shape
        return jnp.einsum('ab,kf->akbf', eye, w).reshape(BB * k, BB * f)

    def tileb(bvec):
        return jnp.tile(bvec, (BB,)).reshape(1, -1)

    e0 = edge_index[:E]
    order = jnp.argsort(e0[:, 1], stable=True)
    gs = jax.nn.one_hot(e0[order, 0], P, dtype=f32)

    node = jnp.arange(P)
    cellb = (jnp.take(rows_emb, node // 9, axis=0) @ pre_W0[EMBED:2 * EMBED]
             + jnp.take(cols_emb, node % 9, axis=0) @ pre_W0[2 * EMBED:]
             + pre_b0)
    cellb_t = jnp.tile(cellb, (1, BB))
    emb_proj = init_emb @ pre_W0[:EMBED]
    embk = jnp.einsum('ab,vf->avbf', eye, emb_proj).reshape(BB * 10, S)
    rmat = jnp.repeat(eye, 10, axis=1)

    prek = jnp.stack([kron(pre_Wr[i]) for i in range(3)])
    preb = jnp.stack([tileb(pre_br[i])[0] for i in range(3)])
    w0sk = kron(msg_W0[:LINEAR])
    w0dk = kron(msg_W0[LINEAR:])
    msgb0 = tileb(msg_b0)
    msgk = jnp.stack([kron(msg_Wr[i]) for i in range(3)])
    msgb = jnp.stack([tileb(msg_br[i])[0] for i in range(3)])
    postak = kron(post_W0[:MSG])
    postxk = kron(post_W0[MSG:])
    postb0 = tileb(post_b0)
    postk = jnp.stack([kron(post_Wr[i]) for i in range(3)])
    postb = jnp.stack([tileb(post_br[i])[0] for i in range(3)])

    w4 = lstm_Wih.reshape(LINEAR, 4, LSTM)
    wihk = jnp.einsum('ab,kgf->akgbf', eye, w4).reshape(S, 4 * S)
    h4 = lstm_Whh.reshape(LSTM, 4, LSTM)
    whhk = jnp.einsum('ab,kgf->akgbf', eye, h4).reshape(S, 4 * S)
    lb4 = (lstm_bih + lstm_bhh).reshape(4, 1, LSTM)
    lstmb = jnp.broadcast_to(lb4, (4, BB, LSTM)).reshape(1, 4 * S)

    outk = kron(out_W)
    outb = tileb(out_b)

    xt = x.reshape(G, BB, P).transpose(0, 2, 1).astype(f32)

    full = lambda arr: pl.BlockSpec(arr.shape, lambda i: (0,) * arr.ndim)
    weights = [gs, rmat, embk, cellb_t, prek, preb, w0sk, w0dk, msgb0,
               msgk, msgb, postak, postxk, postb0, postk, postb,
               wihk, whhk, lstmb, outk, outb]
    in_specs = [pl.BlockSpec((1, P, BB), lambda i: (i, 0, 0))]
    in_specs += [full(w) for w in weights]

    out = pl.pallas_call(
        _rrn_block,
        grid=(G,),
        in_specs=in_specs,
        out_specs=pl.BlockSpec((1, P, BB * 9), lambda i: (i, 0, 0)),
        out_shape=jax.ShapeDtypeStruct((G, P, BB * 9), f32),
        compiler_params=pltpu.CompilerParams(
            dimension_semantics=("arbitrary",)),
    )(xt, *weights)

    return out.reshape(G, P, BB, 9).transpose(0, 2, 1, 3).reshape(batch, P, 9)
